# baseline (speedup 1.0000x reference)
import functools

import jax
import jax.numpy as jnp
from jax import lax
from jax.experimental import pallas as pl
from jax.experimental.pallas import tpu as pltpu
from jax.experimental.pallas import tpu_sc as plsc

_N = 10000
_E = 320000
_NC = 2
_NS = 16
_NW = _NC * _NS
_EPW = _E // _NW
_CH = 125
_NCH = _EPW // _CH
_RPT = 624
_RTAIL = _N - _NS * _RPT
_DEGW = 8


def _sc_mesh():
    return plsc.VectorSubcoreMesh(core_axis_name="c", subcore_axis_name="s")


@functools.cache
def _make_prop(D: int):
    @functools.partial(
        pl.kernel,
        mesh=_sc_mesh(),
        out_type=jax.ShapeDtypeStruct((_NC, _N, D), jnp.float32),
        compiler_params=pltpu.CompilerParams(use_tc_tiling_on_sc=False),
        scratch_types=[
            pltpu.VMEM((_NCH, _CH), jnp.int32),
            pltpu.VMEM((_NCH, _CH), jnp.int32),
            [pltpu.VMEM((_CH, D), jnp.float32) for _ in range(4)],
            [pltpu.SemaphoreType.DMA for _ in range(4)],
            [pltpu.SemaphoreType.DMA for _ in range(4)],
            pltpu.VMEM_SHARED((_N, D), jnp.float32),
        ],
    )
    def prop(y_hbm, em_hbm, zero_hbm, out_hbm,
             srcm_v, dstm_v, rows, gsem, ssem, acc_sh):
        c = lax.axis_index("c")
        s = lax.axis_index("s")
        wid = s * _NC + c
        rbase = s * _RPT
        pltpu.sync_copy(zero_hbm.at[pl.ds(rbase, _RPT)],
                        acc_sh.at[pl.ds(rbase, _RPT)])

        @pl.when(s == _NS - 1)
        def _():
            pltpu.sync_copy(zero_hbm.at[pl.ds(_NS * _RPT, _RTAIL)],
                            acc_sh.at[pl.ds(_NS * _RPT, _RTAIL)])

        pltpu.sync_copy(em_hbm.at[0, pl.ds(wid * _NCH, _NCH)], srcm_v)
        pltpu.sync_copy(em_hbm.at[1, pl.ds(wid * _NCH, _NCH)], dstm_v)
        plsc.subcore_barrier()

        def gather(ci, b):
            pltpu.async_copy(y_hbm.at[srcm_v.at[ci]], rows[b], gsem[b])

        def wait_gather(ci, b):
            pltpu.make_async_copy(y_hbm.at[srcm_v.at[ci]], rows[b],
                                  gsem[b]).wait()

        def scatter(ci, b):
            pltpu.async_copy(rows[b], acc_sh.at[dstm_v.at[ci]], ssem[b],
                             add=True)

        def wait_scatter(ci, b):
            pltpu.make_async_copy(rows[b], acc_sh.at[dstm_v.at[ci]],
                                  ssem[b]).wait()

        gather(0, 0)
        gather(1, 1)
        NB = _NCH // 4

        def body(j, carry):
            c0 = 4 * j

            for b in range(4):
                ci = c0 + b
                wait_gather(ci, b)
                scatter(ci, b)
                if b < 2:
                    @pl.when(j > 0)
                    def _(b=b, ci=ci):
                        wait_scatter(ci - 2, (b + 2) % 4)
                    gather(ci + 2, (b + 2) % 4)
                else:
                    @pl.when(j < NB - 1)
                    def _(b=b, ci=ci):
                        wait_scatter(ci - 2, (b + 2) % 4)
                        gather(ci + 2, (b + 2) % 4)
            return carry

        lax.fori_loop(0, NB, body, 0)
        wait_scatter(_NCH - 4, 0)
        wait_scatter(_NCH - 3, 1)
        wait_scatter(_NCH - 2, 2)
        wait_scatter(_NCH - 1, 3)

        plsc.subcore_barrier()
        pltpu.sync_copy(acc_sh.at[pl.ds(rbase, _RPT)],
                        out_hbm.at[c, pl.ds(rbase, _RPT)])

        @pl.when(s == _NS - 1)
        def _():
            pltpu.sync_copy(acc_sh.at[pl.ds(_NS * _RPT, _RTAIL)],
                            out_hbm.at[c, pl.ds(_NS * _RPT, _RTAIL)])

    return prop


@functools.cache
def _make_deg():
    @functools.partial(
        pl.kernel,
        mesh=_sc_mesh(),
        out_type=jax.ShapeDtypeStruct((_NC, _N, _DEGW), jnp.float32),
        compiler_params=pltpu.CompilerParams(use_tc_tiling_on_sc=False),
        scratch_types=[
            pltpu.VMEM((_NCH, _CH), jnp.int32),
            pltpu.VMEM((_CH, _DEGW), jnp.float32),
            pltpu.VMEM_SHARED((_N, _DEGW), jnp.float32),
            pltpu.SemaphoreType.DMA,
        ],
    )
    def degk(em_hbm, ones_hbm, zero_hbm, out_hbm, dstm_v, ones_v, acc_sh, sem):
        c = lax.axis_index("c")
        s = lax.axis_index("s")
        wid = s * _NC + c
        rbase = s * _RPT
        pltpu.sync_copy(zero_hbm.at[pl.ds(rbase, _RPT)],
                        acc_sh.at[pl.ds(rbase, _RPT)])

        @pl.when(s == _NS - 1)
        def _():
            pltpu.sync_copy(zero_hbm.at[pl.ds(_NS * _RPT, _RTAIL)],
                            acc_sh.at[pl.ds(_NS * _RPT, _RTAIL)])

        pltpu.sync_copy(ones_hbm, ones_v)
        pltpu.sync_copy(em_hbm.at[1, pl.ds(wid * _NCH, _NCH)], dstm_v)
        plsc.subcore_barrier()

        def body(i, carry):
            pltpu.async_copy(ones_v, acc_sh.at[dstm_v.at[i]], sem, add=True)
            return carry

        lax.fori_loop(0, _NCH, body, 0)

        def drain(i, carry):
            pltpu.make_async_copy(ones_v, acc_sh.at[dstm_v.at[i]], sem).wait()
            return carry

        lax.fori_loop(0, _NCH, drain, 0)
        plsc.subcore_barrier()
        pltpu.sync_copy(acc_sh.at[pl.ds(rbase, _RPT)],
                        out_hbm.at[c, pl.ds(rbase, _RPT)])

        @pl.when(s == _NS - 1)
        def _():
            pltpu.sync_copy(acc_sh.at[pl.ds(_NS * _RPT, _RTAIL)],
                            out_hbm.at[c, pl.ds(_NS * _RPT, _RTAIL)])

    return degk


def _deg_call(em):
    ones = jnp.ones((_CH, _DEGW), jnp.float32)
    zero = jnp.zeros((_N, _DEGW), jnp.float32)
    return _make_deg()(em, ones, zero)


def _prop_call(y, em):
    D = y.shape[1]
    zero = jnp.zeros((_N, D), jnp.float32)
    return _make_prop(D)(y, em, zero)


_R = 2000


def _kA1(x, W1):
    Din, Dh = W1.shape

    def body(x_ref, w_ref, xw_ref):
        xw_ref[...] = jnp.dot(x_ref[...], w_ref[...],
                              preferred_element_type=jnp.float32)

    return pl.pallas_call(
        body,
        grid=(_N // _R,),
        in_specs=[
            pl.BlockSpec((_R, Din), lambda i: (i, 0)),
            pl.BlockSpec((Din, Dh), lambda i: (0, 0)),
        ],
        out_specs=pl.BlockSpec((_R, Dh), lambda i: (i, 0)),
        out_shape=jax.ShapeDtypeStruct((_N, Dh), jnp.float32),
    )(x, W1)


def _kB(ps, xw, dis, b, Wn):
    Dh = xw.shape[1]
    Dn = Wn.shape[1]

    def body(ps_ref, xw_ref, dis_ref, b_ref, w_ref, xwn_ref):
        dis = dis_ref[...]
        h = jnp.maximum(
            dis * ps_ref[...] + dis * dis * xw_ref[...] + b_ref[...], 0.0)
        xwn_ref[...] = jnp.dot(h, w_ref[...],
                               preferred_element_type=jnp.float32)

    return pl.pallas_call(
        body,
        grid=(_N // _R,),
        in_specs=[
            pl.BlockSpec((_R, Dh), lambda i: (i, 0)),
            pl.BlockSpec((_R, Dh), lambda i: (i, 0)),
            pl.BlockSpec((_R, 1), lambda i: (i, 0)),
            pl.BlockSpec((Dh,), lambda i: (0,)),
            pl.BlockSpec((Dh, Dn), lambda i: (0, 0)),
        ],
        out_specs=pl.BlockSpec((_R, Dn), lambda i: (i, 0)),
        out_shape=jax.ShapeDtypeStruct((_N, Dn), jnp.float32),
    )(ps, xw, dis, b, Wn)


def _kC(ps, xw, dis, b):
    Dz = xw.shape[1]

    def body(ps_ref, xw_ref, dis_ref, b_ref, z_ref):
        dis = dis_ref[...]
        z_ref[...] = jnp.maximum(
            dis * ps_ref[...] + dis * dis * xw_ref[...] + b_ref[...], 0.0)

    return pl.pallas_call(
        body,
        grid=(_N // _R,),
        in_specs=[
            pl.BlockSpec((_R, Dz), lambda i: (i, 0)),
            pl.BlockSpec((_R, Dz), lambda i: (i, 0)),
            pl.BlockSpec((_R, 1), lambda i: (i, 0)),
            pl.BlockSpec((Dz,), lambda i: (0,)),
        ],
        out_specs=pl.BlockSpec((_R, Dz), lambda i: (i, 0)),
        out_shape=jax.ShapeDtypeStruct((_N, Dz), jnp.float32),
    )(ps, xw, dis, b)


def _kD(qs, z, dis, W3, b3, W5, b5):
    Dz = z.shape[1]
    Dh = W3.shape[1]

    def body(qs_ref, z_ref, dis_ref, w3_ref, b3_ref, w5_ref, b5_ref,
             a_ref, s5_ref):
        dis = dis_ref[...]
        t = dis * qs_ref[...] + dis * dis * z_ref[...]
        a_ref[...] = jnp.maximum(
            jnp.dot(t, w3_ref[...], preferred_element_type=jnp.float32)
            + b3_ref[...], 0.0)
        s5_ref[...] = jnp.maximum(
            jnp.dot(t, w5_ref[...], preferred_element_type=jnp.float32)
            + b5_ref[...], 0.0)

    return pl.pallas_call(
        body,
        grid=(_N // _R,),
        in_specs=[
            pl.BlockSpec((_R, Dz), lambda i: (i, 0)),
            pl.BlockSpec((_R, Dz), lambda i: (i, 0)),
            pl.BlockSpec((_R, 1), lambda i: (i, 0)),
            pl.BlockSpec((Dz, Dh), lambda i: (0, 0)),
            pl.BlockSpec((Dh,), lambda i: (0,)),
            pl.BlockSpec((Dz, Dh), lambda i: (0, 0)),
            pl.BlockSpec((Dh,), lambda i: (0,)),
        ],
        out_specs=[
            pl.BlockSpec((_R, Dh), lambda i: (i, 0)),
            pl.BlockSpec((_R, Dh), lambda i: (i, 0)),
        ],
        out_shape=[
            jax.ShapeDtypeStruct((_N, Dh), jnp.float32),
            jax.ShapeDtypeStruct((_N, Dh), jnp.float32),
        ],
    )(qs, z, dis, W3, b3, W5, b5)


def _kE(rs, a, dis, W4, b4):
    Dh, Dout = W4.shape

    def body(rs_ref, a_ref, dis_ref, w_ref, b_ref, o_ref):
        dis = dis_ref[...]
        u = dis * rs_ref[...] + dis * dis * a_ref[...]
        o_ref[...] = jnp.maximum(
            jnp.dot(u, w_ref[...], preferred_element_type=jnp.float32)
            + b_ref[...], 0.0)

    return pl.pallas_call(
        body,
        grid=(_N // _R,),
        in_specs=[
            pl.BlockSpec((_R, Dh), lambda i: (i, 0)),
            pl.BlockSpec((_R, Dh), lambda i: (i, 0)),
            pl.BlockSpec((_R, 1), lambda i: (i, 0)),
            pl.BlockSpec((Dh, Dout), lambda i: (0, 0)),
            pl.BlockSpec((Dout,), lambda i: (0,)),
        ],
        out_specs=pl.BlockSpec((_R, Dout), lambda i: (i, 0)),
        out_shape=jax.ShapeDtypeStruct((_N, Dout), jnp.float32),
    )(rs, a, dis, W4, b4)


_RB = 400


def _kF(s5):
    Dh = s5.shape[1]

    def body(a_ref, b_ref, o_ref):
        logits = lax.dot_general(
            a_ref[...], b_ref[...],
            dimension_numbers=(((1,), (1,)), ((), ())),
            preferred_element_type=jnp.float32)
        o_ref[...] = jax.nn.sigmoid(logits)

    return pl.pallas_call(
        body,
        grid=(_N // _RB,),
        in_specs=[
            pl.BlockSpec((_RB, Dh), lambda i: (i, 0)),
            pl.BlockSpec((_N, Dh), lambda i: (0, 0)),
        ],
        out_specs=pl.BlockSpec((_RB, _N), lambda i: (i, 0)),
        out_shape=jax.ShapeDtypeStruct((_N, _N), jnp.float32),
    )(s5, s5)


def kernel(x, edge_index, W1, b1, W2, b2, W3, b3, W4, b4, W5, b5):
    em = edge_index.astype(jnp.int32).reshape(2, _E // _CH, _CH)

    xw1 = _kA1(x, W1)
    degp = _deg_call(em)
    dis = lax.rsqrt(degp[0, :, 0:1] + degp[1, :, 0:1] + 1.0)
    p1 = _prop_call(dis * xw1, em)
    xw2 = _kB(p1[0] + p1[1], xw1, dis, b1, W2)
    p2 = _prop_call(dis * xw2, em)
    z = _kC(p2[0] + p2[1], xw2, dis, b2)
    q = _prop_call(dis * z, em)
    a, s5 = _kD(q[0] + q[1], z, dis, W3, b3, W5, b5)
    r = _prop_call(dis * a, em)
    adj_recon = _kF(s5)
    x_recon = _kE(r[0] + r[1], a, dis, W4, b4)
    return (x_recon, adj_recon, z)


# speedup vs baseline: 1.0612x; 1.0612x over previous
import functools

import jax
import jax.numpy as jnp
from jax import lax
from jax.experimental import pallas as pl
from jax.experimental.pallas import tpu as pltpu
from jax.experimental.pallas import tpu_sc as plsc

_N = 10000
_E = 320000
_NC = 2
_NS = 16
_NW = _NC * _NS
_EPW = _E // _NW
_CH = 125
_NCH = _EPW // _CH
_RPT = 624
_RTAIL = _N - _NS * _RPT
_DEGW = 8


def _sc_mesh():
    return plsc.VectorSubcoreMesh(core_axis_name="c", subcore_axis_name="s")


@functools.cache
def _make_prop(D: int):
    @functools.partial(
        pl.kernel,
        mesh=_sc_mesh(),
        out_type=jax.ShapeDtypeStruct((_NC, _N, D), jnp.float32),
        compiler_params=pltpu.CompilerParams(use_tc_tiling_on_sc=False),
        scratch_types=[
            pltpu.VMEM((_NCH, _CH), jnp.int32),
            pltpu.VMEM((_NCH, _CH), jnp.int32),
            [pltpu.VMEM((_CH, D), jnp.float32) for _ in range(4)],
            [pltpu.SemaphoreType.DMA for _ in range(4)],
            [pltpu.SemaphoreType.DMA for _ in range(4)],
            pltpu.VMEM_SHARED((_N, D), jnp.float32),
        ],
    )
    def prop(y_hbm, em_hbm, zero_hbm, out_hbm,
             srcm_v, dstm_v, rows, gsem, ssem, acc_sh):
        c = lax.axis_index("c")
        s = lax.axis_index("s")
        wid = s * _NC + c
        rbase = s * _RPT
        pltpu.sync_copy(zero_hbm.at[pl.ds(rbase, _RPT)],
                        acc_sh.at[pl.ds(rbase, _RPT)])

        @pl.when(s == _NS - 1)
        def _():
            pltpu.sync_copy(zero_hbm.at[pl.ds(_NS * _RPT, _RTAIL)],
                            acc_sh.at[pl.ds(_NS * _RPT, _RTAIL)])

        pltpu.sync_copy(em_hbm.at[0, pl.ds(wid * _NCH, _NCH)], srcm_v)
        pltpu.sync_copy(em_hbm.at[1, pl.ds(wid * _NCH, _NCH)], dstm_v)
        plsc.subcore_barrier()

        def gather(ci, b):
            pltpu.async_copy(y_hbm.at[srcm_v.at[ci]], rows[b], gsem[b])

        def wait_gather(ci, b):
            pltpu.make_async_copy(y_hbm.at[srcm_v.at[ci]], rows[b],
                                  gsem[b]).wait()

        def scatter(ci, b):
            pltpu.async_copy(rows[b], acc_sh.at[dstm_v.at[ci]], ssem[b],
                             add=True)

        def wait_scatter(ci, b):
            pltpu.make_async_copy(rows[b], acc_sh.at[dstm_v.at[ci]],
                                  ssem[b]).wait()

        gather(0, 0)
        gather(1, 1)
        NB = _NCH // 4

        def body(j, carry):
            c0 = 4 * j

            for b in range(4):
                ci = c0 + b
                wait_gather(ci, b)
                scatter(ci, b)
                if b < 2:
                    @pl.when(j > 0)
                    def _(b=b, ci=ci):
                        wait_scatter(ci - 2, (b + 2) % 4)
                    gather(ci + 2, (b + 2) % 4)
                else:
                    @pl.when(j < NB - 1)
                    def _(b=b, ci=ci):
                        wait_scatter(ci - 2, (b + 2) % 4)
                        gather(ci + 2, (b + 2) % 4)
            return carry

        lax.fori_loop(0, NB, body, 0)
        wait_scatter(_NCH - 4, 0)
        wait_scatter(_NCH - 3, 1)
        wait_scatter(_NCH - 2, 2)
        wait_scatter(_NCH - 1, 3)

        plsc.subcore_barrier()
        pltpu.sync_copy(acc_sh.at[pl.ds(rbase, _RPT)],
                        out_hbm.at[c, pl.ds(rbase, _RPT)])

        @pl.when(s == _NS - 1)
        def _():
            pltpu.sync_copy(acc_sh.at[pl.ds(_NS * _RPT, _RTAIL)],
                            out_hbm.at[c, pl.ds(_NS * _RPT, _RTAIL)])

    return prop


@functools.cache
def _make_deg():
    @functools.partial(
        pl.kernel,
        mesh=_sc_mesh(),
        out_type=jax.ShapeDtypeStruct((_NC, _N, _DEGW), jnp.float32),
        compiler_params=pltpu.CompilerParams(use_tc_tiling_on_sc=False),
        scratch_types=[
            pltpu.VMEM((_NCH, _CH), jnp.int32),
            pltpu.VMEM((_CH, _DEGW), jnp.float32),
            pltpu.VMEM_SHARED((_N, _DEGW), jnp.float32),
            pltpu.SemaphoreType.DMA,
        ],
    )
    def degk(em_hbm, ones_hbm, zero_hbm, out_hbm, dstm_v, ones_v, acc_sh, sem):
        c = lax.axis_index("c")
        s = lax.axis_index("s")
        wid = s * _NC + c
        rbase = s * _RPT
        pltpu.sync_copy(zero_hbm.at[pl.ds(rbase, _RPT)],
                        acc_sh.at[pl.ds(rbase, _RPT)])

        @pl.when(s == _NS - 1)
        def _():
            pltpu.sync_copy(zero_hbm.at[pl.ds(_NS * _RPT, _RTAIL)],
                            acc_sh.at[pl.ds(_NS * _RPT, _RTAIL)])

        pltpu.sync_copy(ones_hbm, ones_v)
        pltpu.sync_copy(em_hbm.at[1, pl.ds(wid * _NCH, _NCH)], dstm_v)
        plsc.subcore_barrier()

        def body(i, carry):
            pltpu.async_copy(ones_v, acc_sh.at[dstm_v.at[i]], sem, add=True)
            return carry

        lax.fori_loop(0, _NCH, body, 0)

        def drain(i, carry):
            pltpu.make_async_copy(ones_v, acc_sh.at[dstm_v.at[i]], sem).wait()
            return carry

        lax.fori_loop(0, _NCH, drain, 0)
        plsc.subcore_barrier()
        pltpu.sync_copy(acc_sh.at[pl.ds(rbase, _RPT)],
                        out_hbm.at[c, pl.ds(rbase, _RPT)])

        @pl.when(s == _NS - 1)
        def _():
            pltpu.sync_copy(acc_sh.at[pl.ds(_NS * _RPT, _RTAIL)],
                            out_hbm.at[c, pl.ds(_NS * _RPT, _RTAIL)])

    return degk


def _deg_call(em):
    ones = jnp.ones((_CH, _DEGW), jnp.float32)
    zero = jnp.zeros((_N, _DEGW), jnp.float32)
    return _make_deg()(em, ones, zero)


def _prop_call(y, em):
    D = y.shape[1]
    zero = jnp.zeros((_N, D), jnp.float32)
    return _make_prop(D)(y, em, zero)


_R = 2000


def _kA1(x, W1):
    Din, Dh = W1.shape

    def body(x_ref, w_ref, xw_ref):
        xw_ref[...] = jnp.dot(x_ref[...], w_ref[...],
                              preferred_element_type=jnp.float32)

    return pl.pallas_call(
        body,
        grid=(_N // _R,),
        in_specs=[
            pl.BlockSpec((_R, Din), lambda i: (i, 0)),
            pl.BlockSpec((Din, Dh), lambda i: (0, 0)),
        ],
        out_specs=pl.BlockSpec((_R, Dh), lambda i: (i, 0)),
        out_shape=jax.ShapeDtypeStruct((_N, Dh), jnp.float32),
    )(x, W1)


def _kA2(degp, xw1):
    Dh = xw1.shape[1]

    def body(degp_ref, xw_ref, dis_ref, y_ref):
        deg = degp_ref[0, :, 0:1] + degp_ref[1, :, 0:1] + 1.0
        dis = lax.rsqrt(deg)
        dis_ref[...] = dis
        y_ref[...] = dis * xw_ref[...]

    return pl.pallas_call(
        body,
        grid=(_N // _R,),
        in_specs=[
            pl.BlockSpec((_NC, _R, _DEGW), lambda i: (0, i, 0)),
            pl.BlockSpec((_R, Dh), lambda i: (i, 0)),
        ],
        out_specs=[
            pl.BlockSpec((_R, 1), lambda i: (i, 0)),
            pl.BlockSpec((_R, Dh), lambda i: (i, 0)),
        ],
        out_shape=[
            jax.ShapeDtypeStruct((_N, 1), jnp.float32),
            jax.ShapeDtypeStruct((_N, Dh), jnp.float32),
        ],
    )(degp, xw1)


def _kB(p, xw, dis, b, Wn):
    Dh = xw.shape[1]
    Dn = Wn.shape[1]

    def body(p_ref, xw_ref, dis_ref, b_ref, w_ref, xwn_ref, yn_ref):
        dis = dis_ref[...]
        h = jnp.maximum(
            dis * (p_ref[0] + p_ref[1]) + dis * dis * xw_ref[...] + b_ref[...],
            0.0)
        xwn = jnp.dot(h, w_ref[...], preferred_element_type=jnp.float32)
        xwn_ref[...] = xwn
        yn_ref[...] = dis * xwn

    return pl.pallas_call(
        body,
        grid=(_N // _R,),
        in_specs=[
            pl.BlockSpec((_NC, _R, Dh), lambda i: (0, i, 0)),
            pl.BlockSpec((_R, Dh), lambda i: (i, 0)),
            pl.BlockSpec((_R, 1), lambda i: (i, 0)),
            pl.BlockSpec((Dh,), lambda i: (0,)),
            pl.BlockSpec((Dh, Dn), lambda i: (0, 0)),
        ],
        out_specs=[
            pl.BlockSpec((_R, Dn), lambda i: (i, 0)),
            pl.BlockSpec((_R, Dn), lambda i: (i, 0)),
        ],
        out_shape=[
            jax.ShapeDtypeStruct((_N, Dn), jnp.float32),
            jax.ShapeDtypeStruct((_N, Dn), jnp.float32),
        ],
    )(p, xw, dis, b, Wn)


def _kC(p, xw, dis, b):
    Dz = xw.shape[1]

    def body(p_ref, xw_ref, dis_ref, b_ref, z_ref, y_ref):
        dis = dis_ref[...]
        z = jnp.maximum(
            dis * (p_ref[0] + p_ref[1]) + dis * dis * xw_ref[...] + b_ref[...],
            0.0)
        z_ref[...] = z
        y_ref[...] = dis * z

    return pl.pallas_call(
        body,
        grid=(_N // _R,),
        in_specs=[
            pl.BlockSpec((_NC, _R, Dz), lambda i: (0, i, 0)),
            pl.BlockSpec((_R, Dz), lambda i: (i, 0)),
            pl.BlockSpec((_R, 1), lambda i: (i, 0)),
            pl.BlockSpec((Dz,), lambda i: (0,)),
        ],
        out_specs=[
            pl.BlockSpec((_R, Dz), lambda i: (i, 0)),
            pl.BlockSpec((_R, Dz), lambda i: (i, 0)),
        ],
        out_shape=[
            jax.ShapeDtypeStruct((_N, Dz), jnp.float32),
            jax.ShapeDtypeStruct((_N, Dz), jnp.float32),
        ],
    )(p, xw, dis, b)


def _kD(q, z, dis, W3, b3, W5, b5):
    Dz = z.shape[1]
    Dh = W3.shape[1]

    def body(q_ref, z_ref, dis_ref, w3_ref, b3_ref, w5_ref, b5_ref,
             a_ref, y4_ref, s5_ref):
        dis = dis_ref[...]
        t = dis * (q_ref[0] + q_ref[1]) + dis * dis * z_ref[...]
        a = jnp.maximum(
            jnp.dot(t, w3_ref[...], preferred_element_type=jnp.float32)
            + b3_ref[...], 0.0)
        a_ref[...] = a
        y4_ref[...] = dis * a
        s5_ref[...] = jnp.maximum(
            jnp.dot(t, w5_ref[...], preferred_element_type=jnp.float32)
            + b5_ref[...], 0.0)

    return pl.pallas_call(
        body,
        grid=(_N // _R,),
        in_specs=[
            pl.BlockSpec((_NC, _R, Dz), lambda i: (0, i, 0)),
            pl.BlockSpec((_R, Dz), lambda i: (i, 0)),
            pl.BlockSpec((_R, 1), lambda i: (i, 0)),
            pl.BlockSpec((Dz, Dh), lambda i: (0, 0)),
            pl.BlockSpec((Dh,), lambda i: (0,)),
            pl.BlockSpec((Dz, Dh), lambda i: (0, 0)),
            pl.BlockSpec((Dh,), lambda i: (0,)),
        ],
        out_specs=[
            pl.BlockSpec((_R, Dh), lambda i: (i, 0)),
            pl.BlockSpec((_R, Dh), lambda i: (i, 0)),
            pl.BlockSpec((_R, Dh), lambda i: (i, 0)),
        ],
        out_shape=[
            jax.ShapeDtypeStruct((_N, Dh), jnp.float32),
            jax.ShapeDtypeStruct((_N, Dh), jnp.float32),
            jax.ShapeDtypeStruct((_N, Dh), jnp.float32),
        ],
    )(q, z, dis, W3, b3, W5, b5)


def _kE(r, a, dis, W4, b4):
    Dh, Dout = W4.shape

    def body(r_ref, a_ref, dis_ref, w_ref, b_ref, o_ref):
        dis = dis_ref[...]
        u = dis * (r_ref[0] + r_ref[1]) + dis * dis * a_ref[...]
        o_ref[...] = jnp.maximum(
            jnp.dot(u, w_ref[...], preferred_element_type=jnp.float32)
            + b_ref[...], 0.0)

    return pl.pallas_call(
        body,
        grid=(_N // _R,),
        in_specs=[
            pl.BlockSpec((_NC, _R, Dh), lambda i: (0, i, 0)),
            pl.BlockSpec((_R, Dh), lambda i: (i, 0)),
            pl.BlockSpec((_R, 1), lambda i: (i, 0)),
            pl.BlockSpec((Dh, Dout), lambda i: (0, 0)),
            pl.BlockSpec((Dout,), lambda i: (0,)),
        ],
        out_specs=pl.BlockSpec((_R, Dout), lambda i: (i, 0)),
        out_shape=jax.ShapeDtypeStruct((_N, Dout), jnp.float32),
    )(r, a, dis, W4, b4)


_RB = 400


def _kF(s5):
    Dh = s5.shape[1]

    def body(a_ref, b_ref, o_ref):
        logits = lax.dot_general(
            a_ref[...], b_ref[...],
            dimension_numbers=(((1,), (1,)), ((), ())),
            preferred_element_type=jnp.float32)
        o_ref[...] = jax.nn.sigmoid(logits)

    return pl.pallas_call(
        body,
        grid=(_N // _RB,),
        in_specs=[
            pl.BlockSpec((_RB, Dh), lambda i: (i, 0)),
            pl.BlockSpec((_N, Dh), lambda i: (0, 0)),
        ],
        out_specs=pl.BlockSpec((_RB, _N), lambda i: (i, 0)),
        out_shape=jax.ShapeDtypeStruct((_N, _N), jnp.float32),
    )(s5, s5)


def kernel(x, edge_index, W1, b1, W2, b2, W3, b3, W4, b4, W5, b5):
    em = edge_index.astype(jnp.int32).reshape(2, _E // _CH, _CH)

    xw1 = _kA1(x, W1)
    degp = _deg_call(em)
    dis, y1 = _kA2(degp, xw1)
    p1 = _prop_call(y1, em)
    xw2, y2 = _kB(p1, xw1, dis, b1, W2)
    p2 = _prop_call(y2, em)
    z, y3 = _kC(p2, xw2, dis, b2)
    q = _prop_call(y3, em)
    a, y4, s5 = _kD(q, z, dis, W3, b3, W5, b5)
    r = _prop_call(y4, em)
    adj_recon = _kF(s5)
    x_recon = _kE(r, a, dis, W4, b4)
    return (x_recon, adj_recon, z)
